# baseline (speedup 1.0000x reference)
import functools

import jax
import jax.numpy as jnp
from jax import lax
from jax.experimental import pallas as pl
from jax.experimental.pallas import tpu as pltpu
from jax.experimental.pallas import tpu_sc as plsc

_EPS = 1e-15

_NC, _NS, _L = 2, 16, 16
_NW = _NC * _NS

_B = 4096
_C = 50
_D = 64
_BPW = _B // _NW
_SEGS_PER_CHUNK = 2
_ROWS = _SEGS_PER_CHUNK * _C
_CHUNKS = _BPW // _SEGS_PER_CHUNK
_PAD = 112


_UNROLL = 5


def _accum_chunk(rows_v, acc_v, ch):
    for s in range(_SEGS_PER_CHUNK):
        accs = tuple(rows_v[s * _C, pl.ds(j * _L, _L)]
                     for j in range(_D // _L))

        def c_body(cc, a, s=s):
            c0 = 1 + cc * _UNROLL
            for u in range(_UNROLL):
                a = tuple(a[j] + rows_v[s * _C + c0 + u, pl.ds(j * _L, _L)]
                          for j in range(_D // _L))
            return a

        accs = lax.fori_loop(0, (_C - 1 - 4) // _UNROLL, c_body, accs)
        for c in range(_C - 4, _C):
            accs = tuple(accs[j] + rows_v[s * _C + c, pl.ds(j * _L, _L)]
                         for j in range(_D // _L))
        seg = ch * _SEGS_PER_CHUNK + s
        for j in range(_D // _L):
            acc_v[seg, pl.ds(j * _L, _L)] = accs[j]


def _make_sc_body(comp):
    def _sc_body(Wt, Wc, tt_h, pc_h, ng_h, out,
                 raw_t, raw_v, idx_v, rows_a, rows_b, acc_v, tidx_v, trows_v,
                 sem_a, sem_b):
        cid = lax.axis_index("c")
        sid = lax.axis_index("s")
        wid = sid * _NC + cid
        base = wid * _BPW
        iota = lax.iota(jnp.int32, _L)

        comp_v = jnp.full((_L,), comp, jnp.int32)
        pltpu.sync_copy(tt_h.at[pl.ds(base, _BPW)], raw_t)
        for g in range(_BPW // _L):
            p = g * _L + iota
            vals = plsc.load_gather(raw_t, [p, comp_v])
            tidx_v[pl.ds(g * _L, _L)] = vals
        pltpu.async_copy(Wt.at[tidx_v], trows_v, sem_a).wait()
        pltpu.sync_copy(trows_v, out.at[2, pl.ds(base, _BPW)])

        last = _CHUNKS - 1
        for pair, src in enumerate((pc_h, ng_h)):
            pltpu.sync_copy(src.at[pl.ds(base, _BPW)], raw_v)

            def ext_body(g, carry):
                ch = g // 7
                g2 = g % 7
                p = jnp.minimum(ch * _ROWS + g2 * _L + iota, _BPW * _C - 1)
                vals = plsc.load_gather(raw_v, [p // _C, p % _C, comp_v])
                idx_v[pl.ds(ch * _PAD + g2 * _L, _L)] = vals
                return carry

            lax.fori_loop(0, _CHUNKS * 7, ext_body, 0)

            pltpu.make_async_copy(
                Wc.at[idx_v.at[pl.ds(0, _ROWS)]], rows_a, sem_a).start()

            def half_body(h, carry):
                ch = h * 2
                pltpu.make_async_copy(
                    Wc.at[idx_v.at[pl.ds((ch + 1) * _PAD, _ROWS)]],
                    rows_b, sem_b).start()
                pltpu.make_async_copy(
                    Wc.at[idx_v.at[pl.ds(ch * _PAD, _ROWS)]],
                    rows_a, sem_a).wait()
                _accum_chunk(rows_a, acc_v, ch)
                nxt = jnp.minimum(ch + 2, last)
                pltpu.make_async_copy(
                    Wc.at[idx_v.at[pl.ds(nxt * _PAD, _ROWS)]],
                    rows_a, sem_a).start()
                pltpu.make_async_copy(
                    Wc.at[idx_v.at[pl.ds((ch + 1) * _PAD, _ROWS)]],
                    rows_b, sem_b).wait()
                _accum_chunk(rows_b, acc_v, ch + 1)
                return carry

            lax.fori_loop(0, _CHUNKS // 2, half_body, 0)
            pltpu.make_async_copy(
                Wc.at[idx_v.at[pl.ds(last * _PAD, _ROWS)]],
                rows_a, sem_a).wait()
            pltpu.sync_copy(acc_v, out.at[pair, pl.ds(base, _BPW)])

    return _sc_body


def _make_sc_gather_sums(comp):
    return functools.partial(
        pl.kernel,
        out_type=jax.ShapeDtypeStruct((3, _B, _D), jnp.float32),
        mesh=plsc.VectorSubcoreMesh(core_axis_name="c", subcore_axis_name="s"),
        scratch_types=[
            pltpu.VMEM((_BPW, 3), jnp.int32),
            pltpu.VMEM((_BPW, _C, 3), jnp.int32),
            pltpu.VMEM((_CHUNKS * _PAD,), jnp.int32),
            pltpu.VMEM((_ROWS, _D), jnp.float32),
            pltpu.VMEM((_ROWS, _D), jnp.float32),
            pltpu.VMEM((_BPW, _D), jnp.float32),
            pltpu.VMEM((_BPW,), jnp.int32),
            pltpu.VMEM((_BPW, _D), jnp.float32),
            pltpu.SemaphoreType.DMA,
            pltpu.SemaphoreType.DMA,
        ],
        compiler_params=pltpu.CompilerParams(use_tc_tiling_on_sc=False,
                                             needs_layout_passes=False),
    )(_make_sc_body(comp))


_sc_gather_sums_by_comp = tuple(_make_sc_gather_sums(c) for c in range(3))


def _loss_body(s0_ref, s1_ref, s2_ref, o_ref):
    total = jnp.float32(0.0)
    for s_ref in (s0_ref, s1_ref, s2_ref):
        p = s_ref[0]
        n = s_ref[1]
        t = s_ref[2]
        pos = t * p + _EPS
        neg = (t * n + _EPS) - 1.0
        sp = jnp.maximum(-pos, 0.0) + jnp.log1p(jnp.exp(-jnp.abs(pos)))
        sn = jnp.maximum(neg, 0.0) + jnp.log1p(jnp.exp(-jnp.abs(neg)))
        total = total + (jnp.sum(sp) + jnp.sum(sn))
    o_ref[0, 0] = total / (_B * _D)


_loss_tc = pl.pallas_call(
    _loss_body,
    out_shape=jax.ShapeDtypeStruct((1, 1), jnp.float32),
    out_specs=pl.BlockSpec(memory_space=pltpu.SMEM),
)


def kernel(target_triples, pos_context, neg_context,
           W_target_head, W_target_tail, W_target_rel,
           W_context_head, W_context_tail, W_context_rel):
    tt = target_triples.astype(jnp.int32)
    pc = pos_context.astype(jnp.int32)
    ng = neg_context.astype(jnp.int32)

    sums = []
    for comp, (Wt, Wc) in enumerate((
            (W_target_head, W_context_head),
            (W_target_rel, W_context_rel),
            (W_target_tail, W_context_tail))):
        sums.append(_sc_gather_sums_by_comp[comp](Wt, Wc, tt, pc, ng))

    return _loss_tc(*sums)[0, 0]


# speedup vs baseline: 1.1162x; 1.1162x over previous
import functools

import jax
import jax.numpy as jnp
from jax import lax
from jax.experimental import pallas as pl
from jax.experimental.pallas import tpu as pltpu
from jax.experimental.pallas import tpu_sc as plsc

_EPS = 1e-15

_NC, _NS, _L = 2, 16, 16
_NW = _NC * _NS

_B = 4096
_C = 50
_D = 64
_BPW = _B // _NW
_SEGS_PER_CHUNK = 2
_ROWS = _SEGS_PER_CHUNK * _C
_CHUNKS = _BPW // _SEGS_PER_CHUNK


_UNROLL = 5


def _accum_chunk(rows_v, acc_v, ch):
    for s in range(_SEGS_PER_CHUNK):
        accs = tuple(rows_v[s * _C, pl.ds(j * _L, _L)]
                     for j in range(_D // _L))

        def c_body(cc, a, s=s):
            c0 = 1 + cc * _UNROLL
            for u in range(_UNROLL):
                a = tuple(a[j] + rows_v[s * _C + c0 + u, pl.ds(j * _L, _L)]
                          for j in range(_D // _L))
            return a

        accs = lax.fori_loop(0, (_C - 1 - 4) // _UNROLL, c_body, accs)
        for c in range(_C - 4, _C):
            accs = tuple(accs[j] + rows_v[s * _C + c, pl.ds(j * _L, _L)]
                         for j in range(_D // _L))
        seg = ch * _SEGS_PER_CHUNK + s
        for j in range(_D // _L):
            acc_v[seg, pl.ds(j * _L, _L)] = accs[j]


def _sc_body(Wt, Wc, ctx_idx, tgt_idx, out,
             idx_v, rows_a, rows_b, acc_v, tidx_v, trows_v, sem_a, sem_b):
    cid = lax.axis_index("c")
    sid = lax.axis_index("s")
    wid = sid * _NC + cid
    base = wid * _BPW

    pltpu.sync_copy(tgt_idx.at[wid], tidx_v)
    pltpu.async_copy(Wt.at[tidx_v], trows_v, sem_a).wait()
    pltpu.sync_copy(trows_v, out.at[2, pl.ds(base, _BPW)])

    last = _CHUNKS - 1
    for pair in range(2):
        pltpu.sync_copy(ctx_idx.at[pair, wid], idx_v)
        pltpu.make_async_copy(Wc.at[idx_v.at[0]], rows_a, sem_a).start()

        def half_body(h, carry):
            ch = h * 2
            pltpu.make_async_copy(Wc.at[idx_v.at[ch + 1]], rows_b,
                                  sem_b).start()
            pltpu.make_async_copy(Wc.at[idx_v.at[ch]], rows_a, sem_a).wait()
            _accum_chunk(rows_a, acc_v, ch)
            nxt = jnp.minimum(ch + 2, last)
            pltpu.make_async_copy(Wc.at[idx_v.at[nxt]], rows_a, sem_a).start()
            pltpu.make_async_copy(Wc.at[idx_v.at[ch + 1]], rows_b,
                                  sem_b).wait()
            _accum_chunk(rows_b, acc_v, ch + 1)
            return carry

        lax.fori_loop(0, _CHUNKS // 2, half_body, 0)
        pltpu.make_async_copy(Wc.at[idx_v.at[last]], rows_a, sem_a).wait()
        pltpu.sync_copy(acc_v, out.at[pair, pl.ds(base, _BPW)])


_sc_gather_sums = functools.partial(
    pl.kernel,
    out_type=jax.ShapeDtypeStruct((3, _B, _D), jnp.float32),
    mesh=plsc.VectorSubcoreMesh(core_axis_name="c", subcore_axis_name="s"),
    scratch_types=[
        pltpu.VMEM((_CHUNKS, _ROWS), jnp.int32),
        pltpu.VMEM((_ROWS, _D), jnp.float32),
        pltpu.VMEM((_ROWS, _D), jnp.float32),
        pltpu.VMEM((_BPW, _D), jnp.float32),
        pltpu.VMEM((_BPW,), jnp.int32),
        pltpu.VMEM((_BPW, _D), jnp.float32),
        pltpu.SemaphoreType.DMA,
        pltpu.SemaphoreType.DMA,
    ],
    compiler_params=pltpu.CompilerParams(use_tc_tiling_on_sc=False),
)(_sc_body)


def _loss_body(s0_ref, s1_ref, s2_ref, o_ref):
    total = jnp.float32(0.0)
    for s_ref in (s0_ref, s1_ref, s2_ref):
        p = s_ref[0]
        n = s_ref[1]
        t = s_ref[2]
        pos = t * p + _EPS
        neg = (t * n + _EPS) - 1.0
        sp = jnp.maximum(-pos, 0.0) + jnp.log1p(jnp.exp(-jnp.abs(pos)))
        sn = jnp.maximum(neg, 0.0) + jnp.log1p(jnp.exp(-jnp.abs(neg)))
        total = total + (jnp.sum(sp) + jnp.sum(sn))
    o_ref[0, 0] = total / (_B * _D)


_loss_tc = pl.pallas_call(
    _loss_body,
    out_shape=jax.ShapeDtypeStruct((1, 1), jnp.float32),
    out_specs=pl.BlockSpec(memory_space=pltpu.SMEM),
)


def kernel(target_triples, pos_context, neg_context,
           W_target_head, W_target_tail, W_target_rel,
           W_context_head, W_context_tail, W_context_rel):
    tt = target_triples.astype(jnp.int32)
    pc = pos_context.astype(jnp.int32)
    ng = neg_context.astype(jnp.int32)

    sums = []
    for comp, (Wt, Wc) in enumerate((
            (W_target_head, W_context_head),
            (W_target_rel, W_context_rel),
            (W_target_tail, W_context_tail))):
        ctx_idx = jnp.stack([pc[:, :, comp], ng[:, :, comp]]).reshape(
            2, _NW, _CHUNKS, _ROWS)
        tgt_idx = tt[:, comp].reshape(_NW, _BPW)
        sums.append(_sc_gather_sums(Wt, Wc, ctx_idx, tgt_idx))

    return _loss_tc(*sums)[0, 0]
